# baseline (speedup 1.0000x reference)
import functools

import jax
import jax.numpy as jnp
from jax import lax
from jax.experimental import pallas as pl
from jax.experimental.pallas import tpu as pltpu
from jax.experimental.pallas import tpu_sc as plsc

N, G, T, S = 320000, 128, 16, 256
SCALE = 1000000.0

NC, NS = 2, 16
NW = NC * NS
RW = N // NW
CHUNK = 128
NCHF = RW // CHUNK
TAIL = RW - NCHF * CHUNK
NBUF = 6


def _sc_segment_sum():
    mesh = plsc.VectorSubcoreMesh(core_axis_name="c", subcore_axis_name="s")

    @functools.partial(
        pl.kernel,
        mesh=mesh,
        out_type=jax.ShapeDtypeStruct((NC, S, G), jnp.float32),
        scratch_types=(
            [pltpu.VMEM((CHUNK, G), jnp.float32) for _ in range(NBUF)]
            + [pltpu.VMEM((CHUNK,), jnp.int32) for _ in range(NBUF)]
            + [pltpu.VMEM((TAIL,), jnp.int32)]
            + [pltpu.VMEM((16, G), jnp.float32)]
            + [pltpu.VMEM_SHARED((S, G), jnp.float32)]
            + [pltpu.SemaphoreType.DMA for _ in range(3 * NBUF)]
        ),
    )
    def seg_sum(x_hbm, idx_hbm, out_hbm, *refs):
        x_vs = refs[0:NBUF]
        i_vs = refs[NBUF:2 * NBUF]
        i_tail = refs[2 * NBUF]
        z_v = refs[2 * NBUF + 1]
        acc_sh = refs[2 * NBUF + 2]
        sx = refs[2 * NBUF + 3:3 * NBUF + 3]
        si = refs[3 * NBUF + 3:4 * NBUF + 3]
        sf = refs[4 * NBUF + 3:5 * NBUF + 3]

        cid = lax.axis_index("c")
        sid = lax.axis_index("s")
        wid = cid * NS + sid
        base = wid * RW

        zrow = jnp.zeros((16,), jnp.float32)
        for r in range(16):
            for c8 in range(G // 16):
                z_v[r, pl.ds(c8 * 16, 16)] = zrow
        pltpu.sync_copy(z_v, acc_sh.at[pl.ds(sid * 16, 16)])
        plsc.subcore_barrier()

        def istart(b, ch):
            pltpu.make_async_copy(
                x_hbm.at[pl.ds(base + ch * CHUNK, CHUNK)], x_vs[b],
                sx[b]).start()
            pltpu.make_async_copy(
                idx_hbm.at[wid, pl.ds(ch * CHUNK, CHUNK)], i_vs[b],
                si[b]).start()

        def iwait(b):
            pltpu.make_async_copy(
                x_hbm.at[pl.ds(0, CHUNK)], x_vs[b], sx[b]).wait()
            pltpu.make_async_copy(
                idx_hbm.at[0, pl.ds(0, CHUNK)], i_vs[b], si[b]).wait()

        def fstart(b):
            pltpu.make_async_copy(
                x_vs[b], acc_sh.at[i_vs[b]], sf[b]).start(add=True)

        def fwait(b):
            pltpu.make_async_copy(
                x_vs[b], acc_sh.at[i_vs[b]], sf[b]).wait()

        for b in range(NBUF):
            istart(b, b)

        iwait(0); fstart(0)
        iwait(1); fstart(1)
        iwait(2); fstart(2)
        iwait(3); fstart(3); fwait(0); istart(0, 6)
        iwait(4); fstart(4); fwait(1); istart(1, 7)
        iwait(5); fstart(5); fwait(2); istart(2, 8)

        def body(jj, carry):
            c0 = NBUF * jj
            iwait(0); fstart(0); fwait(3); istart(3, c0 + 3)
            iwait(1); fstart(1); fwait(4); istart(4, c0 + 4)
            iwait(2); fstart(2); fwait(5); istart(5, c0 + 5)
            iwait(3); fstart(3); fwait(0); istart(0, c0 + 6)
            iwait(4); fstart(4); fwait(1); istart(1, c0 + 7)
            iwait(5); fstart(5); fwait(2); istart(2, c0 + 8)
            return carry

        lax.fori_loop(1, NCHF // NBUF - 1, body, 0)

        c0 = NCHF - NBUF
        iwait(0); fstart(0); fwait(3); istart(3, c0 + 3)
        iwait(1); fstart(1); fwait(4); istart(4, c0 + 4)
        iwait(2); fstart(2); fwait(5); istart(5, c0 + 5)
        iwait(3); fstart(3); fwait(0)
        iwait(4); fstart(4); fwait(1)
        iwait(5); fstart(5); fwait(2)
        fwait(3); fwait(4); fwait(5)

        pltpu.make_async_copy(
            x_hbm.at[pl.ds(base + NCHF * CHUNK, TAIL)],
            x_vs[0].at[pl.ds(0, TAIL)], sx[0]).start()
        pltpu.make_async_copy(
            idx_hbm.at[wid, pl.ds(NCHF * CHUNK, TAIL)], i_tail,
            si[0]).start()
        pltpu.make_async_copy(
            x_hbm.at[pl.ds(0, TAIL)], x_vs[0].at[pl.ds(0, TAIL)],
            sx[0]).wait()
        pltpu.make_async_copy(
            idx_hbm.at[0, pl.ds(0, TAIL)], i_tail, si[0]).wait()
        pltpu.sync_copy(x_vs[0].at[pl.ds(0, TAIL)], acc_sh.at[i_tail],
                        add=True)

        plsc.subcore_barrier()
        pltpu.sync_copy(acc_sh.at[pl.ds(sid * 16, 16)],
                        out_hbm.at[cid, pl.ds(sid * 16, 16)])

    return seg_sum


def _tc_finish(p_ref, w_ref, ilr_ref, xb_ref):
    raw = p_ref[0] + p_ref[1]
    rs = jnp.sum(raw, axis=1, keepdims=True)
    xb = raw * (SCALE / jnp.clip(rs, 1e-12, None))
    xb_ref[...] = xb
    ilr_ref[...] = jax.lax.dot_general(
        xb, w_ref[...], (((1,), (1,)), ((), ())),
        preferred_element_type=jnp.float32)


_tc_finish_call = pl.pallas_call(
    _tc_finish,
    out_shape=[
        jax.ShapeDtypeStruct((S, T), jnp.float32),
        jax.ShapeDtypeStruct((S, G), jnp.float32),
    ],
)


def kernel(X_batch, batch_idx, W):
    idx2 = batch_idx.astype(jnp.int32).reshape(NW, RW)
    partials = _sc_segment_sum()(X_batch, idx2)
    ilr_y, X_bulk = _tc_finish_call(partials, W)
    return (ilr_y, X_bulk)
